# Initial kernel scaffold; baseline (speedup 1.0000x reference)
import jax
import jax.numpy as jnp
from jax.experimental import pallas as pl


def kernel(indices, W_q, W_r):
    raise NotImplementedError("write your pallas kernel here")


# speedup vs baseline: 22.2010x; 22.2010x over previous
import functools

import jax
import jax.numpy as jnp
from jax import lax
from jax.experimental import pallas as pl
from jax.experimental.pallas import tpu as pltpu
from jax.experimental.pallas import tpu_sc as plsc

BATCH = 16384
NUM_FIELDS = 26
EMB_DIM = 64
NUM_COLLISIONS = 1000
NUM_Q_ROWS = 1000

N_LOOKUPS = BATCH * NUM_FIELDS
NUM_WORKERS = 32
PER_WORKER = N_LOOKUPS // NUM_WORKERS
CHUNK = 128
N_CHUNKS = PER_WORKER // CHUNK
LANES = 16


def _body(idx_hbm, wq_hbm, wr_hbm, out_hbm,
          raw_v, qidx_v, ridx_v, qrows_v, rrows_v, sem_q, sem_r):
    nc = 2
    wid = lax.axis_index("s") * nc + lax.axis_index("c")
    wbase = wid * PER_WORKER

    iota = lax.iota(jnp.int32, LANES)

    @pl.loop(0, N_CHUNKS)
    def _chunk(c):
        base = wbase + c * CHUNK
        pltpu.sync_copy(idx_hbm.at[pl.ds(base, CHUNK)], raw_v)

        for i in range(CHUNK // LANES):
            raw = raw_v[pl.ds(i * LANES, LANES)]
            n = base + i * LANES + iota
            f = lax.rem(n, NUM_FIELDS)
            q = lax.div(raw, NUM_COLLISIONS)
            r = raw - q * NUM_COLLISIONS
            off = f * NUM_Q_ROWS
            qidx_v[pl.ds(i * LANES, LANES)] = off + q
            ridx_v[pl.ds(i * LANES, LANES)] = off + r

        cq = pltpu.async_copy(wq_hbm.at[qidx_v], qrows_v, sem_q)
        cr = pltpu.async_copy(wr_hbm.at[ridx_v], rrows_v, sem_r)
        cq.wait()
        cr.wait()

        @pl.loop(0, CHUNK, unroll=4)
        def _mul(row):
            qr = qrows_v.at[row]
            rr = rrows_v.at[row]
            for j in range(EMB_DIM // LANES):
                sl = pl.ds(j * LANES, LANES)
                qr[sl] = qr[sl] * rr[sl]

        pltpu.sync_copy(qrows_v, out_hbm.at[pl.ds(base, CHUNK)])


@jax.jit
def kernel(indices, W_q, W_r):
    idx_flat = indices.reshape(-1)
    wq_flat = W_q.reshape(NUM_FIELDS * NUM_Q_ROWS, EMB_DIM)
    wr_flat = W_r.reshape(NUM_FIELDS * NUM_COLLISIONS, EMB_DIM)

    mesh = plsc.VectorSubcoreMesh(core_axis_name="c", subcore_axis_name="s")
    out = pl.kernel(
        _body,
        out_type=jax.ShapeDtypeStruct((N_LOOKUPS, EMB_DIM), jnp.float32),
        mesh=mesh,
        compiler_params=pltpu.CompilerParams(use_tc_tiling_on_sc=False),
        scratch_types=[
            pltpu.VMEM((CHUNK,), jnp.int32),
            pltpu.VMEM((CHUNK,), jnp.int32),
            pltpu.VMEM((CHUNK,), jnp.int32),
            pltpu.VMEM((CHUNK, EMB_DIM), jnp.float32),
            pltpu.VMEM((CHUNK, EMB_DIM), jnp.float32),
            pltpu.SemaphoreType.DMA,
            pltpu.SemaphoreType.DMA,
        ],
    )(idx_flat, wq_flat, wr_flat)
    return out.reshape(BATCH, NUM_FIELDS * EMB_DIM)


# speedup vs baseline: 49.5704x; 2.2328x over previous
import jax
import jax.numpy as jnp
from jax import lax
from jax.experimental import pallas as pl
from jax.experimental.pallas import tpu as pltpu
from jax.experimental.pallas import tpu_sc as plsc

BATCH = 16384
NUM_FIELDS = 26
EMB_DIM = 64
NUM_COLLISIONS = 1000
NUM_Q_ROWS = 1000

N_LOOKUPS = BATCH * NUM_FIELDS
NUM_WORKERS = 32
PER_WORKER = N_LOOKUPS // NUM_WORKERS
CHUNK = 128
N_CHUNKS = PER_WORKER // CHUNK
LANES = 16


def _body(idx_hbm, wq_hbm, wr_hbm, out_hbm,
          raw_v, qidx_v, ridx_v, qrows_v, rrows_v, obuf_v,
          sq0, sq1, sr0, sr1, so0, so1):
    nc = 2
    wid = lax.axis_index("s") * nc + lax.axis_index("c")
    wbase = wid * PER_WORKER

    iota = lax.iota(jnp.int32, LANES)
    sems_q = (sq0, sq1)
    sems_r = (sr0, sr1)
    sems_o = (so0, so1)

    pltpu.sync_copy(idx_hbm.at[pl.ds(wbase, PER_WORKER)], raw_v)

    def prep(c, p):
        base = wbase + c * CHUNK
        qi = qidx_v.at[p]
        ri = ridx_v.at[p]
        for i in range(CHUNK // LANES):
            raw = raw_v[pl.ds(c * CHUNK + i * LANES, LANES)]
            n = base + i * LANES + iota
            f = lax.rem(n, NUM_FIELDS)
            q = lax.div(raw, NUM_COLLISIONS)
            r = raw - q * NUM_COLLISIONS
            off = f * NUM_Q_ROWS
            qi[pl.ds(i * LANES, LANES)] = off + q
            ri[pl.ds(i * LANES, LANES)] = off + r
        pltpu.async_copy(wq_hbm.at[qi], qrows_v.at[p], sems_q[p])
        pltpu.async_copy(wr_hbm.at[ri], rrows_v.at[p], sems_r[p])

    def consume(c, p, wait_write):
        base = wbase + c * CHUNK
        pltpu.make_async_copy(wq_hbm.at[qidx_v.at[p]], qrows_v.at[p], sems_q[p]).wait()
        pltpu.make_async_copy(wr_hbm.at[ridx_v.at[p]], rrows_v.at[p], sems_r[p]).wait()
        if wait_write:
            prev = base - 2 * CHUNK
            pltpu.make_async_copy(
                obuf_v.at[p], out_hbm.at[pl.ds(prev, CHUNK)], sems_o[p]
            ).wait()

        @plsc.parallel_loop(0, CHUNK, step=1, unroll=8)
        def _mul(row):
            qr = qrows_v.at[p].at[row]
            rr = rrows_v.at[p].at[row]
            ob = obuf_v.at[p].at[row]
            sls = [pl.ds(j * LANES, LANES) for j in range(EMB_DIM // LANES)]
            qs = [qr[sl] for sl in sls]
            rs = [rr[sl] for sl in sls]
            for sl, q, r in zip(sls, qs, rs):
                ob[sl] = q * r

        pltpu.async_copy(obuf_v.at[p], out_hbm.at[pl.ds(base, CHUNK)], sems_o[p])

    prep(0, 0)
    prep(1, 1)
    consume(0, 0, False)
    prep(2, 0)
    consume(1, 1, False)
    prep(3, 1)

    @pl.loop(0, N_CHUNKS // 2 - 2)
    def _pair(i):
        c = 2 * i + 2
        consume(c, 0, True)
        prep(c + 2, 0)
        consume(c + 1, 1, True)
        prep(c + 3, 1)

    consume(N_CHUNKS - 2, 0, True)
    consume(N_CHUNKS - 1, 1, True)
    for p, c in ((0, N_CHUNKS - 2), (1, N_CHUNKS - 1)):
        base = wbase + c * CHUNK
        pltpu.make_async_copy(
            obuf_v.at[p], out_hbm.at[pl.ds(base, CHUNK)], sems_o[p]
        ).wait()


@jax.jit
def kernel(indices, W_q, W_r):
    idx_flat = indices.reshape(-1)
    wq_flat = W_q.reshape(NUM_FIELDS * NUM_Q_ROWS, EMB_DIM)
    wr_flat = W_r.reshape(NUM_FIELDS * NUM_COLLISIONS, EMB_DIM)

    mesh = plsc.VectorSubcoreMesh(core_axis_name="c", subcore_axis_name="s")
    out = pl.kernel(
        _body,
        out_type=jax.ShapeDtypeStruct((N_LOOKUPS, EMB_DIM), jnp.float32),
        mesh=mesh,
        compiler_params=pltpu.CompilerParams(use_tc_tiling_on_sc=False),
        scratch_types=[
            pltpu.VMEM((PER_WORKER,), jnp.int32),
            pltpu.VMEM((2, CHUNK), jnp.int32),
            pltpu.VMEM((2, CHUNK), jnp.int32),
            pltpu.VMEM((2, CHUNK, EMB_DIM), jnp.float32),
            pltpu.VMEM((2, CHUNK, EMB_DIM), jnp.float32),
            pltpu.VMEM((2, CHUNK, EMB_DIM), jnp.float32),
            pltpu.SemaphoreType.DMA,
            pltpu.SemaphoreType.DMA,
            pltpu.SemaphoreType.DMA,
            pltpu.SemaphoreType.DMA,
            pltpu.SemaphoreType.DMA,
            pltpu.SemaphoreType.DMA,
        ],
    )(idx_flat, wq_flat, wr_flat)
    return out.reshape(BATCH, NUM_FIELDS * EMB_DIM)
